# baseline (speedup 1.0000x reference)
import functools

import jax
import jax.numpy as jnp
from jax import lax
from jax.experimental import pallas as pl
from jax.experimental.pallas import tpu as pltpu
from jax.experimental.pallas import tpu_sc as plsc

_FOV = 5.0
_SBIN = 25
_C_KMS = 299792.458
_H0 = 70.0
_OMEGA_M = 0.3

_N = 5_000_000
_CHUNK = 2000
_NCHUNKS = _N // _CHUNK
_NWORKERS = 32
_VECS = _CHUNK // 16


def _bin_params(z):
    zs = jnp.linspace(0.0, 1.0, 257) * z
    inv_ez = 1.0 / jnp.sqrt(_OMEGA_M * (1.0 + zs) ** 3 + (1.0 - _OMEGA_M))
    dz = zs[1:] - zs[:-1]
    integ = jnp.sum(0.5 * (inv_ez[1:] + inv_ez[:-1]) * dz)
    d_c = (_C_KMS / _H0) * integ
    d_a = d_c / (1.0 + z)
    ang_kpc_per_arcsec = d_a * 1000.0 * (jnp.pi / (180.0 * 3600.0))
    aperture = _FOV * ang_kpc_per_arcsec
    lo = -aperture / 2.0
    inv_step = _SBIN / aperture
    return lo, inv_step


def _dg(src, idx):
    return lax.gather(
        src, idx[:, None],
        lax.GatherDimensionNumbers(
            offset_dims=(), collapsed_slice_dims=(0,), start_index_map=(0,)),
        slice_sizes=(1,), mode=lax.GatherScatterMode.PROMISE_IN_BOUNDS)


def _sc_body(params_hbm, coords_hbm, out_hbm, pbuf, inbuf, obuf):
    wid = lax.axis_index("s") * 2 + lax.axis_index("c")
    pltpu.sync_copy(params_hbm, pbuf)
    scale = pbuf[pl.ds(0, 16)]
    offs = pbuf[pl.ds(16, 16)]

    k = lax.iota(jnp.int32, 16)
    ixl = (k * 3) & 15
    iyl = (k * 3 + 1) & 15
    mxa, mxb = k <= 5, (k >= 6) & (k <= 10)
    mya, myb = k <= 4, (k >= 5) & (k <= 10)

    nch = jnp.where(wid < _NCHUNKS - 32 * (_NCHUNKS // 32), 1, 0) + _NCHUNKS // 32

    def chunk_body(t, _):
        c = wid + 32 * t
        base = c * _CHUNK
        pltpu.sync_copy(coords_hbm.at[pl.ds(base * 3, _CHUNK * 3)], inbuf)

        def vec_body(v, _):
            a = inbuf[pl.ds(v * 48, 16)]
            b = inbuf[pl.ds(v * 48 + 16, 16)]
            cc = inbuf[pl.ds(v * 48 + 32, 16)]
            x = jnp.where(mxa, _dg(a, ixl), jnp.where(mxb, _dg(b, ixl), _dg(cc, ixl)))
            y = jnp.where(mya, _dg(a, iyl), jnp.where(myb, _dg(b, iyl), _dg(cc, iyl)))
            ix = jnp.clip((x * scale + offs).astype(jnp.int32), 0, _SBIN - 1)
            iy = jnp.clip((y * scale + offs).astype(jnp.int32), 0, _SBIN - 1)
            obuf[pl.ds(v * 16, 16)] = (ix + _SBIN * iy).astype(jnp.float32)
            return 0

        lax.fori_loop(0, _VECS, vec_body, 0)
        pltpu.sync_copy(obuf, out_hbm.at[pl.ds(base, _CHUNK)])
        return 0

    lax.fori_loop(0, nch, chunk_body, 0)


def kernel(coords, galaxy_dist_z):
    z = jnp.squeeze(galaxy_dist_z)
    lo, inv_step = _bin_params(z)
    params = jnp.concatenate([
        jnp.full((16,), inv_step, dtype=jnp.float32),
        jnp.full((16,), -lo * inv_step, dtype=jnp.float32),
    ])
    coords_flat = coords.reshape(-1)

    mesh = plsc.VectorSubcoreMesh(core_axis_name="c", subcore_axis_name="s")
    run = functools.partial(
        pl.kernel,
        mesh=mesh,
        out_type=jax.ShapeDtypeStruct((_N,), jnp.float32),
        scratch_types=[
            pltpu.VMEM((32,), jnp.float32),
            pltpu.VMEM((_CHUNK * 3,), jnp.float32),
            pltpu.VMEM((_CHUNK,), jnp.float32),
        ],
    )(_sc_body)
    return run(params, coords_flat)


# speedup vs baseline: 40.0531x; 40.0531x over previous
import functools

import jax
import jax.numpy as jnp
from jax import lax
from jax.experimental import pallas as pl
from jax.experimental.pallas import tpu as pltpu
from jax.experimental.pallas import tpu_sc as plsc

_FOV = 5.0
_SBIN = 25
_C_KMS = 299792.458
_H0 = 70.0
_OMEGA_M = 0.3

_N = 5_000_000
_CHUNK = 2000
_NCHUNKS = _N // _CHUNK
_NWORKERS = 32
_VECS = _CHUNK // 16


def _bin_params(z):
    zs = jnp.linspace(0.0, 1.0, 257) * z
    inv_ez = 1.0 / jnp.sqrt(_OMEGA_M * (1.0 + zs) ** 3 + (1.0 - _OMEGA_M))
    dz = zs[1:] - zs[:-1]
    integ = jnp.sum(0.5 * (inv_ez[1:] + inv_ez[:-1]) * dz)
    d_c = (_C_KMS / _H0) * integ
    d_a = d_c / (1.0 + z)
    ang_kpc_per_arcsec = d_a * 1000.0 * (jnp.pi / (180.0 * 3600.0))
    aperture = _FOV * ang_kpc_per_arcsec
    lo = -aperture / 2.0
    inv_step = _SBIN / aperture
    return lo, inv_step


def _sc_body(params_hbm, x_hbm, y_hbm, out_hbm, pbuf, xbuf, ybuf, obuf):
    wid = lax.axis_index("s") * 2 + lax.axis_index("c")
    pltpu.sync_copy(params_hbm, pbuf)
    scale = pbuf[pl.ds(0, 16)]
    offs = pbuf[pl.ds(16, 16)]

    nch = jnp.where(wid < _NCHUNKS - 32 * (_NCHUNKS // 32), 1, 0) + _NCHUNKS // 32

    def chunk_body(t, _):
        c = wid + 32 * t
        base = c * _CHUNK
        pltpu.sync_copy(x_hbm.at[pl.ds(base, _CHUNK)], xbuf)
        pltpu.sync_copy(y_hbm.at[pl.ds(base, _CHUNK)], ybuf)

        def vec_body(v, _):
            x = xbuf[pl.ds(v * 16, 16)]
            y = ybuf[pl.ds(v * 16, 16)]
            ix = jnp.clip((x * scale + offs).astype(jnp.int32), 0, _SBIN - 1)
            iy = jnp.clip((y * scale + offs).astype(jnp.int32), 0, _SBIN - 1)
            obuf[pl.ds(v * 16, 16)] = (ix + _SBIN * iy).astype(jnp.float32)
            return 0

        lax.fori_loop(0, _VECS, vec_body, 0)
        pltpu.sync_copy(obuf, out_hbm.at[pl.ds(base, _CHUNK)])
        return 0

    lax.fori_loop(0, nch, chunk_body, 0)


def kernel(coords, galaxy_dist_z):
    z = jnp.squeeze(galaxy_dist_z)
    lo, inv_step = _bin_params(z)
    params = jnp.concatenate([
        jnp.full((16,), inv_step, dtype=jnp.float32),
        jnp.full((16,), -lo * inv_step, dtype=jnp.float32),
    ])
    x = coords[:, 0]
    y = coords[:, 1]

    mesh = plsc.VectorSubcoreMesh(core_axis_name="c", subcore_axis_name="s")
    run = functools.partial(
        pl.kernel,
        mesh=mesh,
        compiler_params=pltpu.CompilerParams(use_tc_tiling_on_sc=True),
        out_type=jax.ShapeDtypeStruct((_N,), jnp.float32),
        scratch_types=[
            pltpu.VMEM((32,), jnp.float32),
            pltpu.VMEM((_CHUNK,), jnp.float32),
            pltpu.VMEM((_CHUNK,), jnp.float32),
            pltpu.VMEM((_CHUNK,), jnp.float32),
        ],
    )(_sc_body)
    return run(params, x, y)
